# baseline (speedup 1.0000x reference)
import jax
import jax.numpy as jnp
from jax import lax
from jax.experimental import pallas as pl
from jax.experimental.pallas import tpu as pltpu
from jax.experimental.pallas import tpu_sc as plsc

NUM_INPUT = 4096
HIDDEN_BATCHES = 23
HIDDEN_SIZE = 4096
NUM_OUTPUT = 1024
FAN_IN = 32
LANES = 16
NUM_TILES = 16
ROWS_HID = HIDDEN_SIZE // NUM_TILES
ROWS_OUT = NUM_OUTPUT // NUM_TILES


def _rows16(wbuf, ibuf, vals, bias_vec, row0, pstart):
    pvec = jnp.full((LANES,), pstart, dtype=jnp.int32)
    prods = []
    for i in range(LANES):
        r = row0 + i
        gi0 = ibuf[r, pl.ds(0, LANES)] - pvec
        gi1 = ibuf[r, pl.ds(LANES, LANES)] - pvec
        w0 = wbuf[r, pl.ds(0, LANES)]
        w1 = wbuf[r, pl.ds(LANES, LANES)]
        g0 = plsc.load_gather(vals, [gi0])
        g1 = plsc.load_gather(vals, [gi1])
        prods.append(w0 * g0 + w1 * g1)
    lane = lax.iota(jnp.int32, LANES)
    d = 1
    while len(prods) > 1:
        pidx = lane ^ d
        m = (lane & d) == 0
        nxt = []
        for k in range(0, len(prods), 2):
            a, b = prods[k], prods[k + 1]
            pa = jnp.take_along_axis(a, pidx, axis=0)
            pb = jnp.take_along_axis(b, pidx, axis=0)
            nxt.append(jnp.where(m, a, pb) + jnp.where(m, pa, b))
        prods = nxt
        d *= 2
    return prods[0] + bias_vec


def _body(x_hbm, hw_hbm, ow_hbm, bias_hbm, hi_hbm, oi_hbm, out_hbm,
          vals, wbuf, ibuf, owbuf, oibuf, bbuf, obuf, shared):
    cid = lax.axis_index("c")
    sid = lax.axis_index("s")

    @pl.when(cid == 0)
    def _():
        base = sid * ROWS_HID
        pltpu.sync_copy(x_hbm, vals)

        def layer(t, carry):
            pltpu.sync_copy(hw_hbm.at[t, pl.ds(base, ROWS_HID), :], wbuf)
            pltpu.sync_copy(hi_hbm.at[t, pl.ds(base, ROWS_HID), :], ibuf)
            pltpu.sync_copy(bias_hbm.at[pl.ds(t * HIDDEN_SIZE + base, ROWS_HID)], bbuf)
            pstart = t * HIDDEN_SIZE

            def rows(r, c2):
                row0 = r * LANES
                bv = bbuf[pl.ds(row0, LANES)]
                a = _rows16(wbuf, ibuf, vals, bv, row0, pstart)
                obuf[pl.ds(row0, LANES)] = a / (1.0 + jnp.exp(-a))
                return c2

            lax.fori_loop(0, ROWS_HID // LANES, rows, 0)

            slot = lax.rem(t, 2)
            pltpu.sync_copy(obuf, shared.at[slot, pl.ds(base, ROWS_HID)])
            plsc.subcore_barrier()
            pltpu.sync_copy(shared.at[slot], vals)
            return carry

        lax.fori_loop(0, HIDDEN_BATCHES, layer, 0)

        base_o = sid * ROWS_OUT
        pltpu.sync_copy(ow_hbm.at[pl.ds(base_o, ROWS_OUT), :], owbuf)
        pltpu.sync_copy(oi_hbm.at[pl.ds(base_o, ROWS_OUT), :], oibuf)
        pltpu.sync_copy(
            bias_hbm.at[pl.ds(HIDDEN_BATCHES * HIDDEN_SIZE + base_o, ROWS_OUT)],
            bbuf.at[pl.ds(0, ROWS_OUT)])
        pstart_o = HIDDEN_BATCHES * HIDDEN_SIZE

        def out_rows(r, c2):
            row0 = r * LANES
            bv = bbuf[pl.ds(row0, LANES)]
            obuf[pl.ds(row0, LANES)] = _rows16(owbuf, oibuf, vals, bv, row0, pstart_o)
            return c2

        lax.fori_loop(0, ROWS_OUT // LANES, out_rows, 0)
        pltpu.sync_copy(obuf.at[pl.ds(0, ROWS_OUT)], out_hbm.at[pl.ds(base_o, ROWS_OUT)])


def kernel(x, hidden_weights, out_weights, bias, hidden_idx, out_idx):
    mesh = plsc.VectorSubcoreMesh(core_axis_name="c", subcore_axis_name="s")
    run = pl.kernel(
        _body,
        mesh=mesh,
        compiler_params=pltpu.CompilerParams(
            needs_layout_passes=False),
        out_type=jax.ShapeDtypeStruct((NUM_OUTPUT,), jnp.float32),
        scratch_types=[
            pltpu.VMEM((HIDDEN_SIZE,), jnp.float32),
            pltpu.VMEM((ROWS_HID, FAN_IN), jnp.float32),
            pltpu.VMEM((ROWS_HID, FAN_IN), jnp.int32),
            pltpu.VMEM((ROWS_OUT, FAN_IN), jnp.float32),
            pltpu.VMEM((ROWS_OUT, FAN_IN), jnp.int32),
            pltpu.VMEM((ROWS_HID,), jnp.float32),
            pltpu.VMEM((ROWS_HID,), jnp.float32),
            pltpu.VMEM_SHARED((2, HIDDEN_SIZE), jnp.float32),
        ],
    )
    return run(x, hidden_weights, out_weights, bias, hidden_idx, out_idx)


# speedup vs baseline: 1.3352x; 1.3352x over previous
import jax
import jax.numpy as jnp
from jax import lax
from jax.experimental import pallas as pl
from jax.experimental.pallas import tpu as pltpu
from jax.experimental.pallas import tpu_sc as plsc

NUM_INPUT = 4096
HIDDEN_BATCHES = 23
HIDDEN_SIZE = 4096
NUM_OUTPUT = 1024
FAN_IN = 32
LANES = 16
NUM_TILES = 16
ROWS_HID = HIDDEN_SIZE // NUM_TILES
ROWS_OUT = NUM_OUTPUT // NUM_TILES
CHUNK = ROWS_HID // 2


def _rows16(wbuf, ibuf, vals, bias_vec, row0, pstart, slot=None):
    pvec = jnp.full((LANES,), pstart, dtype=jnp.int32)
    prods = []
    for i in range(LANES):
        r = row0 + i
        if slot is None:
            gi0 = ibuf[r, pl.ds(0, LANES)] - pvec
            gi1 = ibuf[r, pl.ds(LANES, LANES)] - pvec
            w0 = wbuf[r, pl.ds(0, LANES)]
            w1 = wbuf[r, pl.ds(LANES, LANES)]
        else:
            gi0 = ibuf[slot, r, pl.ds(0, LANES)] - pvec
            gi1 = ibuf[slot, r, pl.ds(LANES, LANES)] - pvec
            w0 = wbuf[slot, r, pl.ds(0, LANES)]
            w1 = wbuf[slot, r, pl.ds(LANES, LANES)]
        g0 = plsc.load_gather(vals, [gi0])
        g1 = plsc.load_gather(vals, [gi1])
        prods.append(w0 * g0 + w1 * g1)
    lane = lax.iota(jnp.int32, LANES)
    d = 1
    while len(prods) > 1:
        pidx = lane ^ d
        m = (lane & d) == 0
        nxt = []
        for k in range(0, len(prods), 2):
            a, b = prods[k], prods[k + 1]
            pa = jnp.take_along_axis(a, pidx, axis=0)
            pb = jnp.take_along_axis(b, pidx, axis=0)
            nxt.append(jnp.where(m, a, pb) + jnp.where(m, pa, b))
        prods = nxt
        d *= 2
    return prods[0] + bias_vec


def _body(x_hbm, hw_hbm, ow_hbm, bias_hbm, hi_hbm, oi_hbm, out_hbm,
          vals, wbuf2, ibuf2, owbuf, oibuf, bbuf2, obbuf, obuf, shared,
          wsem, isem, bsem, osem):
    cid = lax.axis_index("c")
    sid = lax.axis_index("s")

    @pl.when(cid == 0)
    def _():
        base = sid * ROWS_HID
        base_o = sid * ROWS_OUT

        def issue(s, slot):
            t = lax.div(s, 2)
            half = lax.rem(s, 2)
            rbase = base + half * CHUNK
            pltpu.async_copy(hw_hbm.at[pl.ds(t, 1), pl.ds(rbase, CHUNK), :],
                             wbuf2.at[pl.ds(slot, 1)], wsem)
            pltpu.async_copy(hi_hbm.at[pl.ds(t, 1), pl.ds(rbase, CHUNK), :],
                             ibuf2.at[pl.ds(slot, 1)], isem)
            pltpu.async_copy(bias_hbm.at[pl.ds(t * HIDDEN_SIZE + rbase, CHUNK)],
                             bbuf2.at[pl.ds(slot * CHUNK, CHUNK)], bsem)

        def wait(s, slot):
            t = lax.div(s, 2)
            half = lax.rem(s, 2)
            rbase = base + half * CHUNK
            pltpu.make_async_copy(hw_hbm.at[pl.ds(t, 1), pl.ds(rbase, CHUNK), :],
                                  wbuf2.at[pl.ds(slot, 1)], wsem).wait()
            pltpu.make_async_copy(hi_hbm.at[pl.ds(t, 1), pl.ds(rbase, CHUNK), :],
                                  ibuf2.at[pl.ds(slot, 1)], isem).wait()
            pltpu.make_async_copy(bias_hbm.at[pl.ds(t * HIDDEN_SIZE + rbase, CHUNK)],
                                  bbuf2.at[pl.ds(slot * CHUNK, CHUNK)], bsem).wait()

        issue(0, 0)
        pltpu.async_copy(ow_hbm.at[pl.ds(base_o, ROWS_OUT), :], owbuf, osem)
        pltpu.async_copy(oi_hbm.at[pl.ds(base_o, ROWS_OUT), :], oibuf, osem)
        pltpu.async_copy(
            bias_hbm.at[pl.ds(HIDDEN_BATCHES * HIDDEN_SIZE + base_o, ROWS_OUT)],
            obbuf, osem)
        pltpu.sync_copy(x_hbm, vals)

        n_steps = 2 * HIDDEN_BATCHES

        def step(s, carry):
            slot = lax.rem(s, 2)
            t = lax.div(s, 2)
            half = lax.rem(s, 2)
            wait(s, slot)

            @pl.when(s + 1 < n_steps)
            def _():
                issue(s + 1, lax.rem(s + 1, 2))

            pstart = t * HIDDEN_SIZE

            def rows(r, c2):
                row0 = r * LANES
                bv = bbuf2[pl.ds(slot * CHUNK + row0, LANES)]
                a = _rows16(wbuf2, ibuf2, vals, bv, row0, pstart, slot=slot)
                obuf[pl.ds(half * CHUNK + row0, LANES)] = a / (1.0 + jnp.exp(-a))
                return c2

            lax.fori_loop(0, CHUNK // LANES, rows, 0)

            @pl.when(half == 1)
            def _():
                xslot = lax.rem(t, 2)
                pltpu.sync_copy(obuf, shared.at[xslot, pl.ds(base, ROWS_HID)])
                plsc.subcore_barrier()
                pltpu.sync_copy(shared.at[xslot], vals)

            return carry

        lax.fori_loop(0, n_steps, step, 0)

        pltpu.make_async_copy(ow_hbm.at[pl.ds(base_o, ROWS_OUT), :], owbuf, osem).wait()
        pltpu.make_async_copy(oi_hbm.at[pl.ds(base_o, ROWS_OUT), :], oibuf, osem).wait()
        pltpu.make_async_copy(
            bias_hbm.at[pl.ds(HIDDEN_BATCHES * HIDDEN_SIZE + base_o, ROWS_OUT)],
            obbuf, osem).wait()
        pstart_o = HIDDEN_BATCHES * HIDDEN_SIZE

        def out_rows(r, c2):
            row0 = r * LANES
            bv = obbuf[pl.ds(row0, LANES)]
            obuf[pl.ds(row0, LANES)] = _rows16(owbuf, oibuf, vals, bv, row0, pstart_o)
            return c2

        lax.fori_loop(0, ROWS_OUT // LANES, out_rows, 0)
        pltpu.sync_copy(obuf.at[pl.ds(0, ROWS_OUT)], out_hbm.at[pl.ds(base_o, ROWS_OUT)])


def kernel(x, hidden_weights, out_weights, bias, hidden_idx, out_idx):
    mesh = plsc.VectorSubcoreMesh(core_axis_name="c", subcore_axis_name="s")
    run = pl.kernel(
        _body,
        mesh=mesh,
        compiler_params=pltpu.CompilerParams(
            needs_layout_passes=False),
        out_type=jax.ShapeDtypeStruct((NUM_OUTPUT,), jnp.float32),
        scratch_types=[
            pltpu.VMEM((HIDDEN_SIZE,), jnp.float32),
            pltpu.VMEM((2, CHUNK, FAN_IN), jnp.float32),
            pltpu.VMEM((2, CHUNK, FAN_IN), jnp.int32),
            pltpu.VMEM((ROWS_OUT, FAN_IN), jnp.float32),
            pltpu.VMEM((ROWS_OUT, FAN_IN), jnp.int32),
            pltpu.VMEM((2 * CHUNK,), jnp.float32),
            pltpu.VMEM((ROWS_OUT,), jnp.float32),
            pltpu.VMEM((ROWS_HID,), jnp.float32),
            pltpu.VMEM_SHARED((2, HIDDEN_SIZE), jnp.float32),
            pltpu.SemaphoreType.DMA,
            pltpu.SemaphoreType.DMA,
            pltpu.SemaphoreType.DMA,
            pltpu.SemaphoreType.DMA,
        ],
    )
    return run(x, hidden_weights, out_weights, bias, hidden_idx, out_idx)


# speedup vs baseline: 1.3990x; 1.0478x over previous
import jax
import jax.numpy as jnp
from jax import lax
from jax.experimental import pallas as pl
from jax.experimental.pallas import tpu as pltpu
from jax.experimental.pallas import tpu_sc as plsc

NUM_INPUT = 4096
HIDDEN_BATCHES = 23
HIDDEN_SIZE = 4096
NUM_OUTPUT = 1024
FAN_IN = 32
LANES = 16
NUM_CORES = 2
NUM_TILES = 16
NUM_WORKERS = NUM_CORES * NUM_TILES
ROWS_HID = HIDDEN_SIZE // NUM_WORKERS
ROWS_OUT = NUM_OUTPUT // NUM_WORKERS


def _rows16(wbuf, ibuf, vals, bias_vec, row0, pstart, slot):
    pvec = jnp.full((LANES,), pstart, dtype=jnp.int32)
    prods = []
    for i in range(LANES):
        r = row0 + i
        gi0 = ibuf[slot, r, pl.ds(0, LANES)] - pvec
        gi1 = ibuf[slot, r, pl.ds(LANES, LANES)] - pvec
        w0 = wbuf[slot, r, pl.ds(0, LANES)]
        w1 = wbuf[slot, r, pl.ds(LANES, LANES)]
        g0 = plsc.load_gather(vals, [gi0])
        g1 = plsc.load_gather(vals, [gi1])
        prods.append(w0 * g0 + w1 * g1)
    lane = lax.iota(jnp.int32, LANES)
    d = 1
    while len(prods) > 1:
        pidx = lane ^ d
        m = (lane & d) == 0
        nxt = []
        for k in range(0, len(prods), 2):
            a, b = prods[k], prods[k + 1]
            pa = jnp.take_along_axis(a, pidx, axis=0)
            pb = jnp.take_along_axis(b, pidx, axis=0)
            nxt.append(jnp.where(m, a, pb) + jnp.where(m, pa, b))
        prods = nxt
        d *= 2
    return prods[0] + bias_vec


def _body(x_hbm, hw_hbm, ow_hbm, bias_hbm, hi_hbm, oi_hbm, out_hbm, vstage,
          vals, wbuf2, ibuf2, owbuf, oibuf, bbuf2, obbuf, obuf,
          wsem, isem, bsem, osem, vsem, hsem):
    cid = lax.axis_index("c")
    sid = lax.axis_index("s")
    wid = cid * NUM_TILES + sid
    base = wid * ROWS_HID
    base_o = wid * ROWS_OUT

    def issue(t, slot):
        pltpu.async_copy(hw_hbm.at[pl.ds(t, 1), pl.ds(base, ROWS_HID), :],
                         wbuf2.at[pl.ds(slot, 1)], wsem)
        pltpu.async_copy(hi_hbm.at[pl.ds(t, 1), pl.ds(base, ROWS_HID), :],
                         ibuf2.at[pl.ds(slot, 1)], isem)
        pltpu.async_copy(bias_hbm.at[pl.ds(t * HIDDEN_SIZE + base, ROWS_HID)],
                         bbuf2.at[pl.ds(slot * ROWS_HID, ROWS_HID)], bsem)

    def wait(t, slot):
        pltpu.make_async_copy(hw_hbm.at[pl.ds(t, 1), pl.ds(base, ROWS_HID), :],
                              wbuf2.at[pl.ds(slot, 1)], wsem).wait()
        pltpu.make_async_copy(hi_hbm.at[pl.ds(t, 1), pl.ds(base, ROWS_HID), :],
                              ibuf2.at[pl.ds(slot, 1)], isem).wait()
        pltpu.make_async_copy(bias_hbm.at[pl.ds(t * HIDDEN_SIZE + base, ROWS_HID)],
                              bbuf2.at[pl.ds(slot * ROWS_HID, ROWS_HID)], bsem).wait()

    issue(0, 0)
    pltpu.async_copy(ow_hbm.at[pl.ds(base_o, ROWS_OUT), :], owbuf, osem)
    pltpu.async_copy(oi_hbm.at[pl.ds(base_o, ROWS_OUT), :], oibuf, osem)
    pltpu.async_copy(
        bias_hbm.at[pl.ds(HIDDEN_BATCHES * HIDDEN_SIZE + base_o, ROWS_OUT)],
        obbuf, osem)
    pltpu.sync_copy(x_hbm, vals)

    def layer(t, carry):
        slot = lax.rem(t, 2)
        wait(t, slot)

        @pl.when(t + 1 < HIDDEN_BATCHES)
        def _():
            issue(t + 1, lax.rem(t + 1, 2))

        pstart = t * HIDDEN_SIZE

        def rows(r, c2):
            row0 = r * LANES
            bv = bbuf2[pl.ds(slot * ROWS_HID + row0, LANES)]
            a = _rows16(wbuf2, ibuf2, vals, bv, row0, pstart, slot)
            obuf[pl.ds(row0, LANES)] = a / (1.0 + jnp.exp(-a))
            return c2

        lax.fori_loop(0, ROWS_HID // LANES, rows, 0)

        pltpu.async_copy(obuf, vstage.at[slot, pl.ds(base, ROWS_HID)], vsem)
        pltpu.make_async_copy(obuf, vstage.at[slot, pl.ds(base, ROWS_HID)], vsem).wait()
        plsc.subcore_barrier()
        pl.semaphore_signal(hsem, 1, core_index=1 - cid)
        pl.semaphore_wait(hsem, 1)
        pltpu.sync_copy(vstage.at[slot], vals)
        return carry

    lax.fori_loop(0, HIDDEN_BATCHES, layer, 0)

    pltpu.make_async_copy(ow_hbm.at[pl.ds(base_o, ROWS_OUT), :], owbuf, osem).wait()
    pltpu.make_async_copy(oi_hbm.at[pl.ds(base_o, ROWS_OUT), :], oibuf, osem).wait()
    pltpu.make_async_copy(
        bias_hbm.at[pl.ds(HIDDEN_BATCHES * HIDDEN_SIZE + base_o, ROWS_OUT)],
        obbuf, osem).wait()
    pstart_o = HIDDEN_BATCHES * HIDDEN_SIZE

    def out_rows(r, c2):
        row0 = r * LANES
        bv = obbuf[pl.ds(row0, LANES)]
        obuf[pl.ds(row0, LANES)] = _orows16(owbuf, oibuf, vals, bv, row0, pstart_o)
        return c2

    lax.fori_loop(0, ROWS_OUT // LANES, out_rows, 0)
    pltpu.sync_copy(obuf.at[pl.ds(0, ROWS_OUT)], out_hbm.at[pl.ds(base_o, ROWS_OUT)])


def _orows16(wbuf, ibuf, vals, bias_vec, row0, pstart):
    pvec = jnp.full((LANES,), pstart, dtype=jnp.int32)
    prods = []
    for i in range(LANES):
        r = row0 + i
        gi0 = ibuf[r, pl.ds(0, LANES)] - pvec
        gi1 = ibuf[r, pl.ds(LANES, LANES)] - pvec
        w0 = wbuf[r, pl.ds(0, LANES)]
        w1 = wbuf[r, pl.ds(LANES, LANES)]
        g0 = plsc.load_gather(vals, [gi0])
        g1 = plsc.load_gather(vals, [gi1])
        prods.append(w0 * g0 + w1 * g1)
    lane = lax.iota(jnp.int32, LANES)
    d = 1
    while len(prods) > 1:
        pidx = lane ^ d
        m = (lane & d) == 0
        nxt = []
        for k in range(0, len(prods), 2):
            a, b = prods[k], prods[k + 1]
            pa = jnp.take_along_axis(a, pidx, axis=0)
            pb = jnp.take_along_axis(b, pidx, axis=0)
            nxt.append(jnp.where(m, a, pb) + jnp.where(m, pa, b))
        prods = nxt
        d *= 2
    return prods[0] + bias_vec


def kernel(x, hidden_weights, out_weights, bias, hidden_idx, out_idx):
    mesh = plsc.VectorSubcoreMesh(core_axis_name="c", subcore_axis_name="s")
    run = pl.kernel(
        _body,
        mesh=mesh,
        compiler_params=pltpu.CompilerParams(needs_layout_passes=False),
        out_type=[
            jax.ShapeDtypeStruct((NUM_OUTPUT,), jnp.float32),
            jax.ShapeDtypeStruct((2, HIDDEN_SIZE), jnp.float32),
        ],
        scratch_types=[
            pltpu.VMEM((HIDDEN_SIZE,), jnp.float32),
            pltpu.VMEM((2, ROWS_HID, FAN_IN), jnp.float32),
            pltpu.VMEM((2, ROWS_HID, FAN_IN), jnp.int32),
            pltpu.VMEM((ROWS_OUT, FAN_IN), jnp.float32),
            pltpu.VMEM((ROWS_OUT, FAN_IN), jnp.int32),
            pltpu.VMEM((2 * ROWS_HID,), jnp.float32),
            pltpu.VMEM((ROWS_OUT,), jnp.float32),
            pltpu.VMEM((ROWS_HID,), jnp.float32),
            pltpu.SemaphoreType.DMA,
            pltpu.SemaphoreType.DMA,
            pltpu.SemaphoreType.DMA,
            pltpu.SemaphoreType.DMA,
            pltpu.SemaphoreType.DMA,
            pltpu.SemaphoreType.REGULAR,
        ],
    )
    out, _ = run(x, hidden_weights, out_weights, bias, hidden_idx, out_idx)
    return out


# speedup vs baseline: 1.9657x; 1.4051x over previous
import jax
import jax.numpy as jnp
import numpy as np
from jax import lax
from jax.experimental import pallas as pl
from jax.experimental.pallas import tpu as pltpu
from jax.experimental.pallas import tpu_sc as plsc

NUM_INPUT = 4096
HIDDEN_BATCHES = 23
HIDDEN_SIZE = 4096
NUM_OUTPUT = 1024
FAN_IN = 32
LANES = 16
NUM_TILES = 16
ROWS_HID = HIDDEN_SIZE // NUM_TILES
ROWS_OUT = NUM_OUTPUT // NUM_TILES
PACK_MINOR = 128
HID_PROWS = HIDDEN_SIZE * FAN_IN // PACK_MINOR
OUT_PROWS = NUM_OUTPUT * FAN_IN // PACK_MINOR
HID_PCHUNK = HID_PROWS // NUM_TILES
OUT_PCHUNK = OUT_PROWS // NUM_TILES
MASK_HI = np.int32(-65536)
MASK_LO = np.int32(65535)


def _rows16(cbuf, vals, bias_vec, prow0, slot=None):
    prods = []
    for i in range(LANES):
        rr = prow0 + (i >> 2)
        cc = (i & 3) * FAN_IN
        if slot is None:
            c0 = cbuf[rr, pl.ds(cc, LANES)]
            c1 = cbuf[rr, pl.ds(cc + LANES, LANES)]
        else:
            c0 = cbuf[slot, rr, pl.ds(cc, LANES)]
            c1 = cbuf[slot, rr, pl.ds(cc + LANES, LANES)]
        w0 = plsc.bitcast(c0 & MASK_HI, jnp.float32)
        w1 = plsc.bitcast(c1 & MASK_HI, jnp.float32)
        g0 = plsc.load_gather(vals, [c0 & MASK_LO])
        g1 = plsc.load_gather(vals, [c1 & MASK_LO])
        prods.append(w0 * g0 + w1 * g1)
    lane = lax.iota(jnp.int32, LANES)
    d = 1
    while len(prods) > 1:
        pidx = lane ^ d
        m = (lane & d) == 0
        nxt = []
        for k in range(0, len(prods), 2):
            a, b = prods[k], prods[k + 1]
            pa = jnp.take_along_axis(a, pidx, axis=0)
            pb = jnp.take_along_axis(b, pidx, axis=0)
            nxt.append(jnp.where(m, a, pb) + jnp.where(m, pa, b))
        prods = nxt
        d *= 2
    return prods[0] + bias_vec


def _body(x_hbm, ch_hbm, co_hbm, bias_hbm, out_hbm,
          vals, cbuf2, cobuf, bbuf2, obbuf, obuf, shared,
          csem, bsem, osem):
    cid = lax.axis_index("c")
    sid = lax.axis_index("s")

    @pl.when(cid == 0)
    def _():
        base = sid * ROWS_HID
        base_o = sid * ROWS_OUT
        pbase = sid * HID_PCHUNK

        def issue(t, slot):
            pltpu.async_copy(ch_hbm.at[pl.ds(t, 1), pl.ds(pbase, HID_PCHUNK), :],
                             cbuf2.at[pl.ds(slot, 1)], csem)
            pltpu.async_copy(bias_hbm.at[pl.ds(t * HIDDEN_SIZE + base, ROWS_HID)],
                             bbuf2.at[pl.ds(slot * ROWS_HID, ROWS_HID)], bsem)

        def wait(t, slot):
            pltpu.make_async_copy(ch_hbm.at[pl.ds(t, 1), pl.ds(pbase, HID_PCHUNK), :],
                                  cbuf2.at[pl.ds(slot, 1)], csem).wait()
            pltpu.make_async_copy(bias_hbm.at[pl.ds(t * HIDDEN_SIZE + base, ROWS_HID)],
                                  bbuf2.at[pl.ds(slot * ROWS_HID, ROWS_HID)], bsem).wait()

        issue(0, 0)
        pltpu.async_copy(co_hbm.at[pl.ds(sid * OUT_PCHUNK, OUT_PCHUNK), :], cobuf, osem)
        pltpu.async_copy(
            bias_hbm.at[pl.ds(HIDDEN_BATCHES * HIDDEN_SIZE + base_o, ROWS_OUT)],
            obbuf, osem)
        pltpu.sync_copy(x_hbm, vals)

        def layer(t, carry):
            slot = lax.rem(t, 2)
            wait(t, slot)

            @pl.when(t + 1 < HIDDEN_BATCHES)
            def _():
                issue(t + 1, lax.rem(t + 1, 2))

            def rows(r, c2):
                row0 = r * LANES
                bv = bbuf2[pl.ds(slot * ROWS_HID + row0, LANES)]
                a = _rows16(cbuf2, vals, bv, r * 4, slot=slot)
                obuf[pl.ds(row0, LANES)] = a / (1.0 + jnp.exp(-a))
                return c2

            lax.fori_loop(0, ROWS_HID // LANES, rows, 0)

            pltpu.sync_copy(obuf, shared.at[slot, pl.ds(base, ROWS_HID)])
            plsc.subcore_barrier()
            pltpu.sync_copy(shared.at[slot], vals)
            return carry

        lax.fori_loop(0, HIDDEN_BATCHES, layer, 0)

        pltpu.make_async_copy(co_hbm.at[pl.ds(sid * OUT_PCHUNK, OUT_PCHUNK), :],
                              cobuf, osem).wait()
        pltpu.make_async_copy(
            bias_hbm.at[pl.ds(HIDDEN_BATCHES * HIDDEN_SIZE + base_o, ROWS_OUT)],
            obbuf, osem).wait()

        def out_rows(r, c2):
            row0 = r * LANES
            bv = obbuf[pl.ds(row0, LANES)]
            obuf[pl.ds(row0, LANES)] = _rows16(cobuf, vals, bv, r * 4)
            return c2

        lax.fori_loop(0, ROWS_OUT // LANES, out_rows, 0)
        pltpu.sync_copy(obuf.at[pl.ds(0, ROWS_OUT)], out_hbm.at[pl.ds(base_o, ROWS_OUT)])


def _pack(weights, idx, local_start):
    wbits = lax.bitcast_convert_type(
        weights.astype(jnp.bfloat16), jnp.uint16).astype(jnp.uint32)
    ibits = (idx - local_start).astype(jnp.uint32)
    packed = lax.bitcast_convert_type((wbits << 16) | ibits, jnp.int32)
    return packed.reshape(*packed.shape[:-2],
                          packed.shape[-2] * packed.shape[-1] // PACK_MINOR,
                          PACK_MINOR)


def kernel(x, hidden_weights, out_weights, bias, hidden_idx, out_idx):
    pstart = (np.arange(HIDDEN_BATCHES, dtype=np.int32)
              * HIDDEN_SIZE)[:, None, None]
    ch = _pack(hidden_weights, hidden_idx, pstart)
    co = _pack(out_weights, out_idx, HIDDEN_BATCHES * HIDDEN_SIZE)

    mesh = plsc.VectorSubcoreMesh(core_axis_name="c", subcore_axis_name="s")
    run = pl.kernel(
        _body,
        mesh=mesh,
        compiler_params=pltpu.CompilerParams(needs_layout_passes=False),
        out_type=jax.ShapeDtypeStruct((NUM_OUTPUT,), jnp.float32),
        scratch_types=[
            pltpu.VMEM((HIDDEN_SIZE,), jnp.float32),
            pltpu.VMEM((2, HID_PCHUNK, PACK_MINOR), jnp.int32),
            pltpu.VMEM((OUT_PCHUNK, PACK_MINOR), jnp.int32),
            pltpu.VMEM((2 * ROWS_HID,), jnp.float32),
            pltpu.VMEM((ROWS_OUT,), jnp.float32),
            pltpu.VMEM((ROWS_HID,), jnp.float32),
            pltpu.VMEM_SHARED((2, HIDDEN_SIZE), jnp.float32),
            pltpu.SemaphoreType.DMA,
            pltpu.SemaphoreType.DMA,
            pltpu.SemaphoreType.DMA,
        ],
    )
    return run(x, ch, co, bias)


# speedup vs baseline: 1.9777x; 1.0061x over previous
import jax
import jax.numpy as jnp
import numpy as np
from jax import lax
from jax.experimental import pallas as pl
from jax.experimental.pallas import tpu as pltpu
from jax.experimental.pallas import tpu_sc as plsc

NUM_INPUT = 4096
HIDDEN_BATCHES = 23
HIDDEN_SIZE = 4096
NUM_OUTPUT = 1024
FAN_IN = 32
LANES = 16
NUM_TILES = 16
ROWS_HID = HIDDEN_SIZE // NUM_TILES
ROWS_OUT = NUM_OUTPUT // NUM_TILES
PACK_MINOR = 128
HID_PROWS = HIDDEN_SIZE * FAN_IN // PACK_MINOR
OUT_PROWS = NUM_OUTPUT * FAN_IN // PACK_MINOR
HID_PCHUNK = HID_PROWS // NUM_TILES
OUT_PCHUNK = OUT_PROWS // NUM_TILES
MASK_HI = np.int32(-65536)
MASK_LO = np.int32(65535)


def _rows16(cbuf, vals, bias_vec, base_off):
    prods = []
    for i in range(LANES):
        off = base_off + i * FAN_IN
        c0 = cbuf[pl.ds(off, LANES)]
        c1 = cbuf[pl.ds(off + LANES, LANES)]
        w0 = plsc.bitcast(c0 & MASK_HI, jnp.float32)
        w1 = plsc.bitcast(c1 & MASK_HI, jnp.float32)
        g0 = plsc.load_gather(vals, [c0 & MASK_LO])
        g1 = plsc.load_gather(vals, [c1 & MASK_LO])
        prods.append(w0 * g0 + w1 * g1)
    lane = lax.iota(jnp.int32, LANES)
    d = 1
    while len(prods) > 1:
        pidx = lane ^ d
        m = (lane & d) == 0
        nxt = []
        for k in range(0, len(prods), 2):
            a, b = prods[k], prods[k + 1]
            pa = jnp.take_along_axis(a, pidx, axis=0)
            pb = jnp.take_along_axis(b, pidx, axis=0)
            nxt.append(jnp.where(m, a, pb) + jnp.where(m, pa, b))
        prods = nxt
        d *= 2
    return prods[0] + bias_vec


def _body(x_hbm, ch_hbm, co_hbm, bias_hbm, out_hbm,
          vals, cbuf2, cobuf, bbuf2, obbuf, obuf, shared,
          csem, bsem, osem):
    cid = lax.axis_index("c")
    sid = lax.axis_index("s")

    @pl.when(cid == 0)
    def _():
        base = sid * ROWS_HID
        base_o = sid * ROWS_OUT
        cw = ROWS_HID * FAN_IN
        ocw = ROWS_OUT * FAN_IN

        def issue(t, slot):
            pltpu.async_copy(
                ch_hbm.at[pl.ds(t * (HIDDEN_SIZE * FAN_IN) + sid * cw, cw)],
                cbuf2.at[pl.ds(slot * cw, cw)], csem)
            pltpu.async_copy(bias_hbm.at[pl.ds(t * HIDDEN_SIZE + base, ROWS_HID)],
                             bbuf2.at[pl.ds(slot * ROWS_HID, ROWS_HID)], bsem)

        def wait(t, slot):
            pltpu.make_async_copy(
                ch_hbm.at[pl.ds(t * (HIDDEN_SIZE * FAN_IN) + sid * cw, cw)],
                cbuf2.at[pl.ds(slot * cw, cw)], csem).wait()
            pltpu.make_async_copy(bias_hbm.at[pl.ds(t * HIDDEN_SIZE + base, ROWS_HID)],
                                  bbuf2.at[pl.ds(slot * ROWS_HID, ROWS_HID)], bsem).wait()

        issue(0, 0)
        pltpu.async_copy(co_hbm.at[pl.ds(sid * ocw, ocw)], cobuf, osem)
        pltpu.async_copy(
            bias_hbm.at[pl.ds(HIDDEN_BATCHES * HIDDEN_SIZE + base_o, ROWS_OUT)],
            obbuf, osem)
        pltpu.sync_copy(x_hbm, vals)

        def layer(t, carry):
            slot = lax.rem(t, 2)
            wait(t, slot)

            @pl.when(t + 1 < HIDDEN_BATCHES)
            def _():
                issue(t + 1, lax.rem(t + 1, 2))

            def rows(r, c2):
                row0 = r * LANES
                bv = bbuf2[pl.ds(slot * ROWS_HID + row0, LANES)]
                a = _rows16(cbuf2, vals, bv,
                            slot * (ROWS_HID * FAN_IN) + row0 * FAN_IN)
                obuf[pl.ds(row0, LANES)] = a / (1.0 + jnp.exp(-a))
                return c2

            lax.fori_loop(0, ROWS_HID // LANES, rows, 0)

            pltpu.sync_copy(obuf, shared.at[slot, pl.ds(base, ROWS_HID)])
            plsc.subcore_barrier()
            pltpu.sync_copy(shared.at[slot], vals)
            return carry

        lax.fori_loop(0, HIDDEN_BATCHES, layer, 0)

        pltpu.make_async_copy(co_hbm.at[pl.ds(sid * ocw, ocw)], cobuf, osem).wait()
        pltpu.make_async_copy(
            bias_hbm.at[pl.ds(HIDDEN_BATCHES * HIDDEN_SIZE + base_o, ROWS_OUT)],
            obbuf, osem).wait()

        def out_rows(r, c2):
            row0 = r * LANES
            bv = obbuf[pl.ds(row0, LANES)]
            obuf[pl.ds(row0, LANES)] = _rows16(cobuf, vals, bv, row0 * FAN_IN)
            return c2

        lax.fori_loop(0, ROWS_OUT // LANES, out_rows, 0)
        pltpu.sync_copy(obuf.at[pl.ds(0, ROWS_OUT)], out_hbm.at[pl.ds(base_o, ROWS_OUT)])


def _pack(weights, idx, local_start):
    wbits = lax.bitcast_convert_type(
        weights.astype(jnp.bfloat16), jnp.uint16).astype(jnp.uint32)
    ibits = (idx - local_start).astype(jnp.uint32)
    packed = lax.bitcast_convert_type((wbits << 16) | ibits, jnp.int32)
    return packed.reshape(-1)


def kernel(x, hidden_weights, out_weights, bias, hidden_idx, out_idx):
    pstart = (np.arange(HIDDEN_BATCHES, dtype=np.int32)
              * HIDDEN_SIZE)[:, None, None]
    ch = _pack(hidden_weights, hidden_idx, pstart)
    co = _pack(out_weights, out_idx, HIDDEN_BATCHES * HIDDEN_SIZE)

    mesh = plsc.VectorSubcoreMesh(core_axis_name="c", subcore_axis_name="s")
    run = pl.kernel(
        _body,
        mesh=mesh,
        compiler_params=pltpu.CompilerParams(needs_layout_passes=False),
        out_type=jax.ShapeDtypeStruct((NUM_OUTPUT,), jnp.float32),
        scratch_types=[
            pltpu.VMEM((HIDDEN_SIZE,), jnp.float32),
            pltpu.VMEM((2 * ROWS_HID * FAN_IN,), jnp.int32),
            pltpu.VMEM((ROWS_OUT * FAN_IN,), jnp.int32),
            pltpu.VMEM((2 * ROWS_HID,), jnp.float32),
            pltpu.VMEM((ROWS_OUT,), jnp.float32),
            pltpu.VMEM((ROWS_HID,), jnp.float32),
            pltpu.VMEM_SHARED((2, HIDDEN_SIZE), jnp.float32),
            pltpu.SemaphoreType.DMA,
            pltpu.SemaphoreType.DMA,
            pltpu.SemaphoreType.DMA,
        ],
    )
    return run(x, ch, co, bias)
